# baseline (speedup 1.0000x reference)
import functools

import jax
import jax.numpy as jnp
from jax import lax
from jax.experimental import pallas as pl
from jax.experimental.pallas import tpu as pltpu
from jax.experimental.pallas import tpu_sc as plsc

B = 4096
D = 32
H = 128
OUT = 64

_BM = 512


def _sc_gather6(tables, idxs):
    info = plsc.get_sparse_core_info()
    nc, ns = info.num_cores, info.num_subcores
    nw = nc * ns
    bpw = B // nw

    mesh = plsc.VectorSubcoreMesh(core_axis_name="c", subcore_axis_name="s")
    out_type = [jax.ShapeDtypeStruct((B, D), jnp.float32) for _ in range(6)]
    scratch_types = (
        [pltpu.VMEM((bpw,), jnp.int32) for _ in range(6)]
        + [pltpu.VMEM((bpw, D), jnp.float32) for _ in range(6)]
        + [pltpu.SemaphoreType.DMA]
    )

    @functools.partial(
        pl.kernel, mesh=mesh, out_type=out_type, scratch_types=scratch_types,
        compiler_params=pltpu.CompilerParams(use_tc_tiling_on_sc=False),
    )
    def k(*refs):
        t = refs[0:6]
        ix = refs[6:12]
        o = refs[12:18]
        iv = refs[18:24]
        rv = refs[24:30]
        sem = refs[30]
        wid = lax.axis_index("s") * nc + lax.axis_index("c")
        base = wid * bpw
        for j in range(6):
            pltpu.sync_copy(ix[j].at[pl.ds(base, bpw)], iv[j])
        copies = [pltpu.async_copy(t[j].at[iv[j]], rv[j], sem) for j in range(6)]
        for c in copies:
            c.wait()
        for j in range(6):
            pltpu.sync_copy(rv[j], o[j].at[pl.ds(base, bpw)])

    return k(*tables, *idxs)


def _tower_block(e0, e1, e2, W1, b1, W2, b2):
    x = (
        jnp.dot(e0, W1[0:D, :], preferred_element_type=jnp.float32)
        + jnp.dot(e1, W1[D : 2 * D, :], preferred_element_type=jnp.float32)
        + jnp.dot(e2, W1[2 * D : 3 * D, :], preferred_element_type=jnp.float32)
    )
    h = jnp.maximum(x + b1[None, :], 0.0)
    e = jnp.dot(h, W2, preferred_element_type=jnp.float32) + b2[None, :]
    n = jnp.sqrt(jnp.sum(e * e, axis=-1, keepdims=True))
    return e / jnp.maximum(n, 1e-12)


def _tc_kernel(
    ue0, ue1, ue2, ie0, ie1, ie2,
    u_W1, u_b1, u_W2, u_b2, i_W1, i_b1, i_W2, i_b2, inv_t,
    out_ref, u_emb, i_emb,
):
    pid = pl.program_id(0)

    @pl.when(pid == 0)
    def _():
        u_emb[...] = _tower_block(
            ue0[...], ue1[...], ue2[...], u_W1[...], u_b1[...], u_W2[...], u_b2[...]
        )
        i_emb[...] = _tower_block(
            ie0[...], ie1[...], ie2[...], i_W1[...], i_b1[...], i_W2[...], i_b2[...]
        )

    ub = u_emb[pl.ds(pid * _BM, _BM), :]
    scores = lax.dot_general(
        ub, i_emb[...], (((1,), (1,)), ((), ())), preferred_element_type=jnp.float32
    )
    out_ref[...] = scores * inv_t[0, 0]


def kernel(user_id, user_geo, user_age, item_id, item_category, item_brand,
           u_tab_id, u_tab_geo, u_tab_age, i_tab_id, i_tab_cat, i_tab_brand,
           u_W1, u_b1, u_W2, u_b2, i_W1, i_b1, i_W2, i_b2, temperature):
    gathered = _sc_gather6(
        [u_tab_id, u_tab_geo, u_tab_age, i_tab_id, i_tab_cat, i_tab_brand],
        [user_id, user_geo, user_age, item_id, item_category, item_brand],
    )

    inv_t = (1.0 / temperature).astype(jnp.float32).reshape(1, 1)

    full = pl.BlockSpec((B, D), lambda i: (0, 0))
    wspec = lambda shape: pl.BlockSpec(shape, lambda i: tuple(0 for _ in shape))
    grid_spec = pltpu.PrefetchScalarGridSpec(
        num_scalar_prefetch=0,
        grid=(B // _BM,),
        in_specs=[
            full, full, full, full, full, full,
            wspec((3 * D, H)), wspec((H,)), wspec((H, OUT)), wspec((OUT,)),
            wspec((3 * D, H)), wspec((H,)), wspec((H, OUT)), wspec((OUT,)),
            pl.BlockSpec(memory_space=pltpu.SMEM),
        ],
        out_specs=pl.BlockSpec((_BM, B), lambda i: (i, 0)),
        scratch_shapes=[
            pltpu.VMEM((B, OUT), jnp.float32),
            pltpu.VMEM((B, OUT), jnp.float32),
        ],
    )

    scores = pl.pallas_call(
        _tc_kernel,
        grid_spec=grid_spec,
        out_shape=jax.ShapeDtypeStruct((B, B), jnp.float32),
    )(*gathered, u_W1, u_b1, u_W2, u_b2, i_W1, i_b1, i_W2, i_b2, inv_t)

    return scores


# speedup vs baseline: 5.6100x; 5.6100x over previous
import functools

import jax
import jax.numpy as jnp
from jax import lax
from jax.experimental import pallas as pl
from jax.experimental.pallas import tpu as pltpu
from jax.experimental.pallas import tpu_sc as plsc

B = 4096
D = 32
H = 128
OUT = 64

_BM = 512
_CH = 16


def _sc_gather_rows(tables, idxs):
    n = len(tables)
    info = plsc.get_sparse_core_info()
    nc, ns = info.num_cores, info.num_subcores
    nw = nc * ns
    bpw = B // nw

    mesh = plsc.VectorSubcoreMesh(core_axis_name="c", subcore_axis_name="s")
    out_type = [jax.ShapeDtypeStruct((B, D), jnp.float32) for _ in range(n)]
    scratch_types = (
        [pltpu.VMEM((bpw,), jnp.int32) for _ in range(n)]
        + [pltpu.VMEM((bpw, D), jnp.float32) for _ in range(n)]
        + [pltpu.SemaphoreType.DMA]
    )

    @functools.partial(
        pl.kernel, mesh=mesh, out_type=out_type, scratch_types=scratch_types,
        compiler_params=pltpu.CompilerParams(use_tc_tiling_on_sc=False),
    )
    def k(*refs):
        t = refs[0:n]
        ix = refs[n:2 * n]
        o = refs[2 * n:3 * n]
        iv = refs[3 * n:4 * n]
        rv = refs[4 * n:5 * n]
        sem = refs[5 * n]
        wid = lax.axis_index("s") * nc + lax.axis_index("c")
        base = wid * bpw
        for j in range(n):
            pltpu.sync_copy(ix[j].at[pl.ds(base, bpw)], iv[j])
        copies = [pltpu.async_copy(t[j].at[iv[j]], rv[j], sem) for j in range(n)]
        for c in copies:
            c.wait()
        for j in range(n):
            pltpu.sync_copy(rv[j], o[j].at[pl.ds(base, bpw)])

    return k(*tables, *idxs)


def _sc_gather_big(tables_t, idxs):
    n = len(tables_t)
    info = plsc.get_sparse_core_info()
    nc, ns = info.num_cores, info.num_subcores
    nw = nc * ns
    bpw = B // nw
    nch = bpw // _CH

    mesh = plsc.VectorSubcoreMesh(core_axis_name="c", subcore_axis_name="s")
    out_type = [jax.ShapeDtypeStruct((D, B), jnp.float32) for _ in range(n)]
    scratch_types = (
        [pltpu.VMEM((bpw,), jnp.int32) for _ in range(n)]
        + [pltpu.VMEM((_CH, D, 128), jnp.float32)]
        + [pltpu.VMEM((D, bpw), jnp.float32) for _ in range(n)]
        + [pltpu.SemaphoreType.DMA]
    )

    @functools.partial(
        pl.kernel, mesh=mesh, out_type=out_type, scratch_types=scratch_types,
        compiler_params=pltpu.CompilerParams(
            use_tc_tiling_on_sc=True, needs_layout_passes=False
        ),
    )
    def k(*refs):
        t = refs[0:n]
        ix = refs[n:2 * n]
        o = refs[2 * n:3 * n]
        iv = refs[3 * n:4 * n]
        stage = refs[4 * n]
        outT = refs[4 * n + 1:5 * n + 1]
        sem = refs[5 * n + 1]
        wid = lax.axis_index("s") * nc + lax.axis_index("c")
        base = wid * bpw
        for j in range(n):
            pltpu.sync_copy(ix[j].at[pl.ds(base, bpw)], iv[j])

        lanes = lax.iota(jnp.int32, 16)
        for j in range(n):
            def chunk(c, _, j=j):
                v16 = iv[j][pl.ds(c * _CH, 16)]
                for l in range(_CH):
                    col = pl.multiple_of((v16[l] >> 7) * 128, 128)
                    pltpu.async_copy(
                        t[j].at[:, pl.ds(col, 128)], stage.at[l], sem
                    )
                for l in range(_CH):
                    pltpu.make_async_copy(
                        t[j].at[:, pl.ds(0, 128)], stage.at[l], sem
                    ).wait()
                sub = v16 & 127
                for f in range(D):
                    fv = jnp.full((16,), f, jnp.int32)
                    vals = plsc.load_gather(stage, [lanes, fv, sub])
                    outT[j][f, pl.ds(c * _CH, 16)] = vals
                return ()

            lax.fori_loop(0, nch, chunk, ())

        for j in range(n):
            pltpu.sync_copy(outT[j], o[j].at[:, pl.ds(base, bpw)])

    return k(*tables_t, *idxs)


def _tower_block(g0T, g1, g2, W1, b1, W2, b2):
    x = (
        lax.dot_general(g0T, W1[0:D, :], (((0,), (0,)), ((), ())),
                        preferred_element_type=jnp.float32)
        + jnp.dot(g1, W1[D:2 * D, :], preferred_element_type=jnp.float32)
        + jnp.dot(g2, W1[2 * D:3 * D, :], preferred_element_type=jnp.float32)
    )
    h = jnp.maximum(x + b1[None, :], 0.0)
    e = jnp.dot(h, W2, preferred_element_type=jnp.float32) + b2[None, :]
    nrm = jnp.sqrt(jnp.sum(e * e, axis=-1, keepdims=True))
    return e / jnp.maximum(nrm, 1e-12)


def _tc_kernel(
    ug0, ug1, ug2, ig0, ig1, ig2,
    u_W1, u_b1, u_W2, u_b2, i_W1, i_b1, i_W2, i_b2, inv_t,
    out_ref, u_emb, i_emb,
):
    pid = pl.program_id(0)

    @pl.when(pid == 0)
    def _():
        u_emb[...] = _tower_block(
            ug0[...], ug1[...], ug2[...], u_W1[...], u_b1[...], u_W2[...], u_b2[...]
        )
        i_emb[...] = _tower_block(
            ig0[...], ig1[...], ig2[...], i_W1[...], i_b1[...], i_W2[...], i_b2[...]
        )

    ub = u_emb[pl.ds(pid * _BM, _BM), :]
    scores = lax.dot_general(
        ub, i_emb[...], (((1,), (1,)), ((), ())), preferred_element_type=jnp.float32
    )
    out_ref[...] = scores * inv_t[0, 0]


def kernel(user_id, user_geo, user_age, item_id, item_category, item_brand,
           u_tab_id, u_tab_geo, u_tab_age, i_tab_id, i_tab_cat, i_tab_brand,
           u_W1, u_b1, u_W2, u_b2, i_W1, i_b1, i_W2, i_b2, temperature):
    g_small = _sc_gather_rows(
        [u_tab_geo, u_tab_age, i_tab_cat, i_tab_brand],
        [user_geo, user_age, item_category, item_brand],
    )
    g_big = _sc_gather_big(
        [u_tab_id.T, i_tab_id.T], [user_id, item_id],
    )

    inv_t = (1.0 / temperature).astype(jnp.float32).reshape(1, 1)

    fullT = pl.BlockSpec((D, B), lambda i: (0, 0))
    full = pl.BlockSpec((B, D), lambda i: (0, 0))
    wspec = lambda shape: pl.BlockSpec(shape, lambda i: tuple(0 for _ in shape))
    grid_spec = pltpu.PrefetchScalarGridSpec(
        num_scalar_prefetch=0,
        grid=(B // _BM,),
        in_specs=[
            fullT, full, full, fullT, full, full,
            wspec((3 * D, H)), wspec((H,)), wspec((H, OUT)), wspec((OUT,)),
            wspec((3 * D, H)), wspec((H,)), wspec((H, OUT)), wspec((OUT,)),
            pl.BlockSpec(memory_space=pltpu.SMEM),
        ],
        out_specs=pl.BlockSpec((_BM, B), lambda i: (i, 0)),
        scratch_shapes=[
            pltpu.VMEM((B, OUT), jnp.float32),
            pltpu.VMEM((B, OUT), jnp.float32),
        ],
    )

    scores = pl.pallas_call(
        _tc_kernel,
        grid_spec=grid_spec,
        out_shape=jax.ShapeDtypeStruct((B, B), jnp.float32),
    )(g_big[0], g_small[0], g_small[1], g_big[1], g_small[2], g_small[3],
      u_W1, u_b1, u_W2, u_b2, i_W1, i_b1, i_W2, i_b2, inv_t)

    return scores


# speedup vs baseline: 6.6131x; 1.1788x over previous
import functools

import jax
import jax.numpy as jnp
from jax import lax
from jax.experimental import pallas as pl
from jax.experimental.pallas import tpu as pltpu
from jax.experimental.pallas import tpu_sc as plsc

B = 4096
D = 32
H = 128
OUT = 64

_BM = 512
_CH = 16


def _sc_gather_rows(tables, idxs, dep):
    n = len(tables)
    info = plsc.get_sparse_core_info()
    nc, ns = info.num_cores, info.num_subcores
    nw = nc * ns
    bpw = B // nw

    mesh = plsc.VectorSubcoreMesh(core_axis_name="c", subcore_axis_name="s")
    out_type = jax.ShapeDtypeStruct((B, n * D), jnp.float32)
    scratch_types = (
        [pltpu.VMEM((bpw,), jnp.int32) for _ in range(n)]
        + [pltpu.VMEM((bpw, D), jnp.float32) for _ in range(n)]
        + [pltpu.SemaphoreType.DMA]
    )

    @functools.partial(
        pl.kernel, mesh=mesh, out_type=out_type, scratch_types=scratch_types,
        compiler_params=pltpu.CompilerParams(use_tc_tiling_on_sc=False),
    )
    def k(*refs):
        t = refs[0:n]
        ix = refs[n:2 * n]
        o = refs[2 * n + 1]
        iv = refs[2 * n + 2:3 * n + 2]
        rv = refs[3 * n + 2:4 * n + 2]
        sem = refs[4 * n + 2]
        wid = lax.axis_index("s") * nc + lax.axis_index("c")
        base = wid * bpw
        for j in range(n):
            pltpu.sync_copy(ix[j].at[pl.ds(base, bpw)], iv[j])
        copies = [pltpu.async_copy(t[j].at[iv[j]], rv[j], sem) for j in range(n)]
        for c in copies:
            c.wait()
        for j in range(n):
            pltpu.sync_copy(
                rv[j], o.at[pl.ds(base, bpw), pl.ds(j * D, D)]
            )

    return k(*tables, *idxs, dep)


def _sc_gather_big(tables_t, idxs):
    n = len(tables_t)
    info = plsc.get_sparse_core_info()
    nc, ns = info.num_cores, info.num_subcores
    nw = nc * ns
    bpw = B // nw
    nch = bpw // _CH

    mesh = plsc.VectorSubcoreMesh(core_axis_name="c", subcore_axis_name="s")
    out_type = [jax.ShapeDtypeStruct((D, B), jnp.float32) for _ in range(n)]
    scratch_types = (
        [pltpu.VMEM((bpw,), jnp.int32) for _ in range(n)]
        + [pltpu.VMEM((_CH, D, 128), jnp.float32)]
        + [pltpu.VMEM((D, bpw), jnp.float32) for _ in range(n)]
        + [pltpu.SemaphoreType.DMA]
    )

    @functools.partial(
        pl.kernel, mesh=mesh, out_type=out_type, scratch_types=scratch_types,
        compiler_params=pltpu.CompilerParams(
            use_tc_tiling_on_sc=True, needs_layout_passes=False
        ),
    )
    def k(*refs):
        t = refs[0:n]
        ix = refs[n:2 * n]
        o = refs[2 * n:3 * n]
        iv = refs[3 * n:4 * n]
        stage = refs[4 * n]
        outT = refs[4 * n + 1:5 * n + 1]
        sem = refs[5 * n + 1]
        wid = lax.axis_index("s") * nc + lax.axis_index("c")
        base = wid * bpw
        for j in range(n):
            pltpu.sync_copy(ix[j].at[pl.ds(base, bpw)], iv[j])

        lanes = lax.iota(jnp.int32, 16)
        for j in range(n):
            def chunk(c, _, j=j):
                v16 = iv[j][pl.ds(c * _CH, 16)]
                for l in range(_CH):
                    col = pl.multiple_of((v16[l] >> 7) * 128, 128)
                    pltpu.async_copy(
                        t[j].at[:, pl.ds(col, 128)], stage.at[l], sem
                    )
                for l in range(_CH):
                    pltpu.make_async_copy(
                        t[j].at[:, pl.ds(0, 128)], stage.at[l], sem
                    ).wait()
                sub = v16 & 127
                for f in range(D):
                    fv = jnp.full((16,), f, jnp.int32)
                    vals = plsc.load_gather(stage, [lanes, fv, sub])
                    outT[j][f, pl.ds(c * _CH, 16)] = vals
                return ()

            lax.fori_loop(0, nch, chunk, ())

        for j in range(n):
            pltpu.sync_copy(outT[j], o[j].at[:, pl.ds(base, bpw)])

    return k(*tables_t, *idxs)


def _tower_block(g0T, g1, g2, W1, b1, W2, b2):
    x = (
        lax.dot_general(g0T, W1[0:D, :], (((0,), (0,)), ((), ())),
                        preferred_element_type=jnp.float32)
        + jnp.dot(g1, W1[D:2 * D, :], preferred_element_type=jnp.float32)
        + jnp.dot(g2, W1[2 * D:3 * D, :], preferred_element_type=jnp.float32)
    )
    h = jnp.maximum(x + b1[None, :], 0.0)
    e = jnp.dot(h, W2, preferred_element_type=jnp.float32) + b2[None, :]
    nrm = jnp.sqrt(jnp.sum(e * e, axis=-1, keepdims=True))
    return e / jnp.maximum(nrm, 1e-12)


def _tc_kernel(
    ug0, ig0, small,
    u_W1, u_b1, u_W2, u_b2, i_W1, i_b1, i_W2, i_b2, inv_t,
    out_ref, u_emb, i_emb,
):
    pid = pl.program_id(0)

    @pl.when(pid == 0)
    def _():
        s = small[...]
        u_emb[...] = _tower_block(
            ug0[...], s[:, 0:D], s[:, D:2 * D],
            u_W1[...], u_b1[...], u_W2[...], u_b2[...]
        )
        i_emb[...] = _tower_block(
            ig0[...], s[:, 2 * D:3 * D], s[:, 3 * D:4 * D],
            i_W1[...], i_b1[...], i_W2[...], i_b2[...]
        )

    ub = u_emb[pl.ds(pid * _BM, _BM), :]
    scores = lax.dot_general(
        ub, i_emb[...], (((1,), (1,)), ((), ())), preferred_element_type=jnp.float32
    )
    out_ref[...] = scores * inv_t[0, 0]


def kernel(user_id, user_geo, user_age, item_id, item_category, item_brand,
           u_tab_id, u_tab_geo, u_tab_age, i_tab_id, i_tab_cat, i_tab_brand,
           u_W1, u_b1, u_W2, u_b2, i_W1, i_b1, i_W2, i_b2, temperature):
    g_big = _sc_gather_big(
        [u_tab_id.T, i_tab_id.T], [user_id, item_id],
    )
    g_small = _sc_gather_rows(
        [u_tab_geo, u_tab_age, i_tab_cat, i_tab_brand],
        [user_geo, user_age, item_category, item_brand],
        g_big[0],
    )

    inv_t = (1.0 / temperature).astype(jnp.float32).reshape(1, 1)

    fullT = pl.BlockSpec((D, B), lambda i: (0, 0))
    wspec = lambda shape: pl.BlockSpec(shape, lambda i: tuple(0 for _ in shape))
    grid_spec = pltpu.PrefetchScalarGridSpec(
        num_scalar_prefetch=0,
        grid=(B // _BM,),
        in_specs=[
            fullT, fullT, pl.BlockSpec((B, 4 * D), lambda i: (0, 0)),
            wspec((3 * D, H)), wspec((H,)), wspec((H, OUT)), wspec((OUT,)),
            wspec((3 * D, H)), wspec((H,)), wspec((H, OUT)), wspec((OUT,)),
            pl.BlockSpec(memory_space=pltpu.SMEM),
        ],
        out_specs=pl.BlockSpec((_BM, B), lambda i: (i, 0)),
        scratch_shapes=[
            pltpu.VMEM((B, OUT), jnp.float32),
            pltpu.VMEM((B, OUT), jnp.float32),
        ],
    )

    scores = pl.pallas_call(
        _tc_kernel,
        grid_spec=grid_spec,
        out_shape=jax.ShapeDtypeStruct((B, B), jnp.float32),
    )(g_big[0], g_big[1], g_small,
      u_W1, u_b1, u_W2, u_b2, i_W1, i_b1, i_W2, i_b2, inv_t)

    return scores


# speedup vs baseline: 6.9518x; 1.0512x over previous
import functools

import jax
import jax.numpy as jnp
from jax import lax
from jax.experimental import pallas as pl
from jax.experimental.pallas import tpu as pltpu
from jax.experimental.pallas import tpu_sc as plsc

B = 4096
D = 32
H = 128
OUT = 64

_BM = 512
_CH = 16


def _sc_gather_rows(tables, idxs, dep):
    n = len(tables)
    info = plsc.get_sparse_core_info()
    nc, ns = info.num_cores, info.num_subcores
    nw = nc * ns
    bpw = B // nw

    mesh = plsc.VectorSubcoreMesh(core_axis_name="c", subcore_axis_name="s")
    out_type = jax.ShapeDtypeStruct((B, n * D), jnp.float32)
    scratch_types = (
        [pltpu.VMEM((bpw,), jnp.int32) for _ in range(n)]
        + [pltpu.VMEM((bpw, D), jnp.float32) for _ in range(n)]
        + [pltpu.SemaphoreType.DMA]
    )

    @functools.partial(
        pl.kernel, mesh=mesh, out_type=out_type, scratch_types=scratch_types,
        compiler_params=pltpu.CompilerParams(use_tc_tiling_on_sc=False),
    )
    def k(*refs):
        t = refs[0:n]
        ix = refs[n:2 * n]
        o = refs[2 * n + 1]
        iv = refs[2 * n + 2:3 * n + 2]
        rv = refs[3 * n + 2:4 * n + 2]
        sem = refs[4 * n + 2]
        wid = lax.axis_index("s") * nc + lax.axis_index("c")
        base = wid * bpw
        for j in range(n):
            pltpu.sync_copy(ix[j].at[pl.ds(base, bpw)], iv[j])
        copies = [pltpu.async_copy(t[j].at[iv[j]], rv[j], sem) for j in range(n)]
        for c in copies:
            c.wait()
        for j in range(n):
            pltpu.sync_copy(
                rv[j], o.at[pl.ds(base, bpw), pl.ds(j * D, D)]
            )

    return k(*tables, *idxs, dep)


def _sc_gather_big(tables_t, idxs):
    n = len(tables_t)
    info = plsc.get_sparse_core_info()
    nc, ns = info.num_cores, info.num_subcores
    nw = nc * ns
    bpw = B // nw
    ch = 8
    nch = bpw // ch

    mesh = plsc.VectorSubcoreMesh(core_axis_name="c", subcore_axis_name="s")
    out_type = [jax.ShapeDtypeStruct((D, B), jnp.float32) for _ in range(n)]
    scratch_types = (
        [pltpu.VMEM((bpw + 8, ), jnp.int32) for _ in range(n)]
        + [pltpu.VMEM((ch, D, 128), jnp.float32) for _ in range(2)]
        + [pltpu.VMEM((D, bpw + 8), jnp.float32)]
        + [pltpu.SemaphoreType.DMA]
    )

    @functools.partial(
        pl.kernel, mesh=mesh, out_type=out_type, scratch_types=scratch_types,
        compiler_params=pltpu.CompilerParams(
            use_tc_tiling_on_sc=True, needs_layout_passes=False
        ),
    )
    def k(*refs):
        t = refs[0:n]
        ix = refs[n:2 * n]
        o = refs[2 * n:3 * n]
        iv = refs[3 * n:4 * n]
        stages = refs[4 * n:4 * n + 2]
        outT = refs[4 * n + 2]
        sem = refs[4 * n + 3]
        wid = lax.axis_index("s") * nc + lax.axis_index("c")
        base = wid * bpw
        for j in range(n):
            pltpu.sync_copy(ix[j].at[pl.ds(base, bpw)], iv[j].at[pl.ds(0, bpw)])

        lanes = lax.iota(jnp.int32, 16)
        lanes8 = lanes & 7

        def fire(j, c, buf):
            v = iv[j][pl.ds(c * ch, 16)]
            for l in range(ch):
                col = pl.multiple_of((v[l] >> 7) * 128, 128)
                pltpu.async_copy(t[j].at[:, pl.ds(col, 128)], buf.at[l], sem)

        def drain(j, buf):
            for l in range(ch):
                pltpu.make_async_copy(
                    t[j].at[:, pl.ds(0, 128)], buf.at[l], sem
                ).wait()

        def extract(j, c, buf):
            v = iv[j][pl.ds(c * ch, 16)]
            sub = v & 127
            for f in range(D):
                fv = jnp.full((16,), f, jnp.int32)
                vals = plsc.load_gather(buf, [lanes8, fv, sub])
                outT[f, pl.ds(c * ch, 16)] = vals

        for j in range(n):
            fire(j, 0, stages[0])

            def pair(cp, _, j=j):
                c0 = cp * 2
                fire(j, c0 + 1, stages[1])
                drain(j, stages[0])
                extract(j, c0, stages[0])

                @pl.when(c0 + 2 < nch)
                def _():
                    fire(j, c0 + 2, stages[0])

                drain(j, stages[1])
                extract(j, c0 + 1, stages[1])
                return ()

            lax.fori_loop(0, nch // 2, pair, ())
            pltpu.sync_copy(
                outT.at[:, pl.ds(0, bpw)], o[j].at[:, pl.ds(base, bpw)]
            )

    return k(*tables_t, *idxs)


def _tower_block(g0T, g1, g2, W1, b1, W2, b2):
    x = (
        lax.dot_general(g0T, W1[0:D, :], (((0,), (0,)), ((), ())),
                        preferred_element_type=jnp.float32)
        + jnp.dot(g1, W1[D:2 * D, :], preferred_element_type=jnp.float32)
        + jnp.dot(g2, W1[2 * D:3 * D, :], preferred_element_type=jnp.float32)
    )
    h = jnp.maximum(x + b1[None, :], 0.0)
    e = jnp.dot(h, W2, preferred_element_type=jnp.float32) + b2[None, :]
    nrm = jnp.sqrt(jnp.sum(e * e, axis=-1, keepdims=True))
    return e / jnp.maximum(nrm, 1e-12)


def _tc_kernel(
    ug0, ig0, small,
    u_W1, u_b1, u_W2, u_b2, i_W1, i_b1, i_W2, i_b2, inv_t,
    out_ref, u_emb, i_emb,
):
    pid = pl.program_id(0)

    @pl.when(pid == 0)
    def _():
        s = small[...]
        u_emb[...] = _tower_block(
            ug0[...], s[:, 0:D], s[:, D:2 * D],
            u_W1[...], u_b1[...], u_W2[...], u_b2[...]
        )
        i_emb[...] = _tower_block(
            ig0[...], s[:, 2 * D:3 * D], s[:, 3 * D:4 * D],
            i_W1[...], i_b1[...], i_W2[...], i_b2[...]
        )

    ub = u_emb[pl.ds(pid * _BM, _BM), :]
    scores = lax.dot_general(
        ub, i_emb[...], (((1,), (1,)), ((), ())), preferred_element_type=jnp.float32
    )
    out_ref[...] = scores * inv_t[0, 0]


def kernel(user_id, user_geo, user_age, item_id, item_category, item_brand,
           u_tab_id, u_tab_geo, u_tab_age, i_tab_id, i_tab_cat, i_tab_brand,
           u_W1, u_b1, u_W2, u_b2, i_W1, i_b1, i_W2, i_b2, temperature):
    g_big = _sc_gather_big(
        [u_tab_id.T, i_tab_id.T], [user_id, item_id],
    )
    g_small = _sc_gather_rows(
        [u_tab_geo, u_tab_age, i_tab_cat, i_tab_brand],
        [user_geo, user_age, item_category, item_brand],
        g_big[0],
    )

    inv_t = (1.0 / temperature).astype(jnp.float32).reshape(1, 1)

    fullT = pl.BlockSpec((D, B), lambda i: (0, 0))
    wspec = lambda shape: pl.BlockSpec(shape, lambda i: tuple(0 for _ in shape))
    grid_spec = pltpu.PrefetchScalarGridSpec(
        num_scalar_prefetch=0,
        grid=(B // _BM,),
        in_specs=[
            fullT, fullT, pl.BlockSpec((B, 4 * D), lambda i: (0, 0)),
            wspec((3 * D, H)), wspec((H,)), wspec((H, OUT)), wspec((OUT,)),
            wspec((3 * D, H)), wspec((H,)), wspec((H, OUT)), wspec((OUT,)),
            pl.BlockSpec(memory_space=pltpu.SMEM),
        ],
        out_specs=pl.BlockSpec((_BM, B), lambda i: (i, 0)),
        scratch_shapes=[
            pltpu.VMEM((B, OUT), jnp.float32),
            pltpu.VMEM((B, OUT), jnp.float32),
        ],
    )

    scores = pl.pallas_call(
        _tc_kernel,
        grid_spec=grid_spec,
        out_shape=jax.ShapeDtypeStruct((B, B), jnp.float32),
    )(g_big[0], g_big[1], g_small,
      u_W1, u_b1, u_W2, u_b2, i_W1, i_b1, i_W2, i_b2, inv_t)

    return scores
